# baseline (speedup 1.0000x reference)
import jax
import jax.numpy as jnp
from jax import lax
from jax.experimental import pallas as pl
from jax.experimental.pallas import tpu as pltpu, tpu_sc as plsc

N = 10000
E = 320000
F = 128

NC = 2
NS = 16
L = 16
NW = NC * NS
EPW = E // NW
NG = EPW // L
RING = 5
CBITS = 14
CMASK = (1 << CBITS) - 1


def _dense_body(x_ref, w_ref, as_ref, an_ref, e_ref,
                f_ref, s_ref, n_ref, rc_ref):
    f = jnp.dot(x_ref[...], w_ref[...], preferred_element_type=jnp.float32)
    f_ref[...] = f
    s_ref[...] = jnp.dot(f, as_ref[...], preferred_element_type=jnp.float32)
    n_ref[...] = jnp.dot(f, an_ref[...], preferred_element_type=jnp.float32)
    rc_ref[...] = (e_ref[0] << CBITS) | e_ref[1]


def _dense(X, W, a_self, a_neigh, e3d):
    return pl.pallas_call(
        _dense_body,
        out_shape=[
            jax.ShapeDtypeStruct((N, F), jnp.float32),
            jax.ShapeDtypeStruct((N, 1), jnp.float32),
            jax.ShapeDtypeStruct((N, 1), jnp.float32),
            jax.ShapeDtypeStruct((E // F, F), jnp.int32),
        ],
    )(X, W, a_self, a_neigh, e3d)


def _sc_body(feat_hbm, s_hbm, n_hbm, rc_hbm, zd_hbm,
             part_hbm, den_hbm,
             s_v, n_v, rc_v, buf, exst, out_sp, den_sp, *sems):
    gsems = sems[:RING]
    ssems = sems[RING:2 * RING]
    dsems = sems[2 * RING:]
    cid = lax.axis_index("c")
    tid = lax.axis_index("s")
    wid = tid * NC + cid
    NPT = N // NS

    pltpu.sync_copy(s_hbm, s_v)
    pltpu.sync_copy(n_hbm, n_v)
    pltpu.sync_copy(rc_hbm.at[wid], rc_v)
    zv = jnp.zeros((L,), jnp.float32)
    for k in range(RING * L):
        for c in range(F // L):
            buf[k, pl.ds(c * L, L)] = zv
    for b in range(RING):
        exst[b, :] = zv
    for st in list(range(0, NPT - RING * L, RING * L)) + [NPT - RING * L]:
        pltpu.sync_copy(buf, out_sp.at[pl.ds(tid * NPT + st, RING * L)])

    @pl.when(tid == 0)
    def _():
        pltpu.sync_copy(zd_hbm, den_sp)

    def _mx(i, m):
        return jnp.maximum(m, n_v[pl.ds(i * L, L)])
    m = lax.fori_loop(0, N // L, _mx, jnp.full((L,), -jnp.inf, jnp.float32))
    lanes = lax.iota(jnp.int32, L)
    for sh in (8, 4, 2, 1):
        m = jnp.maximum(m, m[jnp.bitwise_xor(lanes, sh)])
    max_n = m

    plsc.subcore_barrier()

    zi = jnp.zeros((L,), jnp.int32)
    for b in range(RING - 1):
        pltpu.async_copy(feat_hbm.at[rc_v[b] & CMASK],
                         buf.at[pl.ds(b * L, L)], gsems[b])
    pltpu.async_copy(buf.at[pl.ds((RING - 1) * L, L)], out_sp.at[zi],
                     ssems[RING - 1], add=True)
    for b in range(RING):
        pltpu.async_copy(exst.at[b], den_sp.at[zi], dsems[b], add=True)

    def _group(o, b, prefetch):
        g = o * RING + b
        pk = rc_v[g]
        rg = pk >> CBITS
        cg = pk & CMASK
        pltpu.make_async_copy(
            feat_hbm.at[cg], buf.at[pl.ds(b * L, L)], gsems[b]).wait()
        sr = plsc.load_gather(s_v, [rg])
        nc_ = plsc.load_gather(n_v, [cg])
        e = sr + nc_
        e = jnp.where(e > 0, e, 0.2 * e)
        cap = sr + max_n
        cap = jnp.where(cap > 0, cap, 0.2 * cap)
        ex = jnp.exp(e - cap)

        pltpu.make_async_copy(exst.at[b], den_sp.at[rg], dsems[b]).wait()
        exst[b, :] = ex
        pltpu.async_copy(exst.at[b], den_sp.at[rg], dsems[b], add=True)

        for j in range(L):
            w = ex[jnp.full((L,), j, jnp.int32)]
            for c in range(F // L):
                sl = slice(c * L, (c + 1) * L)
                buf[b * L + j, sl] = buf[b * L + j, sl] * w

        pltpu.async_copy(buf.at[pl.ds(b * L, L)], out_sp.at[rg],
                         ssems[b], add=True)

        if prefetch:
            bp = (b - 1) % RING
            pg = g + RING - 1
            pltpu.make_async_copy(
                buf.at[pl.ds(bp * L, L)], out_sp.at[rg], ssems[bp]).wait()
            pltpu.async_copy(feat_hbm.at[rc_v[pg] & CMASK],
                             buf.at[pl.ds(bp * L, L)], gsems[bp])

    def _outer(o, carry):
        for b in range(RING):
            _group(o, b, prefetch=True)
        return carry

    lax.fori_loop(0, NG // RING - 1, _outer, jnp.int32(0))
    for b in range(RING):
        _group(NG // RING - 1, b, prefetch=(b == 0))

    for b in range(RING):
        pltpu.make_async_copy(
            buf.at[pl.ds(b * L, L)], out_sp.at[zi], ssems[b]).wait()
        pltpu.make_async_copy(exst.at[b], den_sp.at[zi], dsems[b]).wait()

    plsc.subcore_barrier()

    pltpu.sync_copy(out_sp.at[pl.ds(tid * NPT, NPT)],
                    part_hbm.at[cid, pl.ds(tid * NPT, NPT)])

    @pl.when(tid == 0)
    def _():
        pltpu.sync_copy(den_sp, den_hbm.at[pl.ds(cid * N, N)])


def _sc_edges(feat, s, n, rc3d, zden):
    mesh = plsc.VectorSubcoreMesh(core_axis_name="c", subcore_axis_name="s")
    scratch = [
        pltpu.VMEM((N,), jnp.float32),
        pltpu.VMEM((N,), jnp.float32),
        pltpu.VMEM((NG, L), jnp.int32),
        pltpu.VMEM((RING * L, F), jnp.float32),
        pltpu.VMEM((RING, L), jnp.float32),
        pltpu.VMEM_SHARED((N, F), jnp.float32),
        pltpu.VMEM_SHARED((N,), jnp.float32),
    ] + [pltpu.SemaphoreType.DMA] * (3 * RING)
    run = pl.kernel(
        _sc_body,
        out_type=[
            jax.ShapeDtypeStruct((NC, N, F), jnp.float32),
            jax.ShapeDtypeStruct((NC * N,), jnp.float32),
        ],
        mesh=mesh,
        scratch_types=scratch,
        compiler_params=pltpu.CompilerParams(
            needs_layout_passes=False, use_tc_tiling_on_sc=False),
    )
    return run(feat, s, n, rc3d, zden)


def _combine_body(p_ref, d_ref, b_ref, o_ref):
    ps = p_ref[0] + p_ref[1]
    den = d_ref[0] + d_ref[1]
    o_ref[...] = jnp.maximum(ps / (den[:, None] + 1e-9) + b_ref[...], 0.0)


def _combine(partials, denoms, bias2d):
    return pl.pallas_call(
        _combine_body,
        out_shape=jax.ShapeDtypeStruct((N, F), jnp.float32),
    )(partials, denoms, bias2d)


def kernel(X, edge_index, W, a_self, a_neigh, bias):
    e3d = edge_index.reshape(2, E // F, F)
    feat, s2, n2, rc = _dense(X, W, a_self, a_neigh, e3d)
    s = s2.reshape(N)
    n = n2.reshape(N)
    rc3d = rc.reshape(NW, NG, L)
    zden = jnp.zeros((N,), jnp.float32)
    partials, denoms = _sc_edges(feat, s, n, rc3d, zden)
    return _combine(partials, denoms.reshape(NC, N), bias.reshape(1, F))


# speedup vs baseline: 1.0016x; 1.0016x over previous
import jax
import jax.numpy as jnp
from jax import lax
from jax.experimental import pallas as pl
from jax.experimental.pallas import tpu as pltpu, tpu_sc as plsc

N = 10000
E = 320000
F = 128

NC = 2
NS = 16
L = 16
NW = NC * NS
EPW = E // NW
NG = EPW // L
RING = 5
CBITS = 14
CMASK = (1 << CBITS) - 1


def _dense_body(x_ref, w_ref, as_ref, an_ref, e_ref,
                f_ref, s_ref, n_ref, rc_ref):
    f = jnp.dot(x_ref[...], w_ref[...], preferred_element_type=jnp.float32)
    f_ref[...] = f.astype(jnp.bfloat16)
    s_ref[...] = jnp.dot(f, as_ref[...], preferred_element_type=jnp.float32)
    n_ref[...] = jnp.dot(f, an_ref[...], preferred_element_type=jnp.float32)
    rc_ref[...] = (e_ref[0] << CBITS) | e_ref[1]


def _dense(X, W, a_self, a_neigh, e3d):
    return pl.pallas_call(
        _dense_body,
        out_shape=[
            jax.ShapeDtypeStruct((N, F), jnp.bfloat16),
            jax.ShapeDtypeStruct((N, 1), jnp.float32),
            jax.ShapeDtypeStruct((N, 1), jnp.float32),
            jax.ShapeDtypeStruct((E // F, F), jnp.int32),
        ],
    )(X, W, a_self, a_neigh, e3d)


_PERM = []
for _c in range(F // 32):
    for _k in range(16):
        _PERM.append(32 * _c + _k)
        _PERM.append(32 * _c + 16 + _k)
_INV = [0] * F
for _p, _col in enumerate(_PERM):
    _INV[_p] = _col


def _sc_body(feat_hbm, s_hbm, n_hbm, rc_hbm, zd_hbm,
             part_hbm, den_hbm,
             s_v, n_v, rc_v, bbuf, sbuf, exst, out_sp, den_sp, *sems):
    gsems = sems[:RING]
    ssems = sems[RING:2 * RING]
    dsems = sems[2 * RING:]
    cid = lax.axis_index("c")
    tid = lax.axis_index("s")
    wid = tid * NC + cid
    NPT = N // NS

    zv = jnp.zeros((L,), jnp.float32)
    for k in range(RING * L):
        for c in range(F // L):
            sbuf[k, pl.ds(c * L, L)] = zv
    for b in range(RING):
        exst[b, :] = zv
    pltpu.sync_copy(s_hbm, s_v)
    pltpu.sync_copy(n_hbm, n_v)
    pltpu.sync_copy(rc_hbm.at[wid], rc_v)
    for st in list(range(0, NPT - RING * L, RING * L)) + [NPT - RING * L]:
        pltpu.sync_copy(sbuf, out_sp.at[pl.ds(tid * NPT + st, RING * L)])

    @pl.when(tid == 0)
    def _():
        pltpu.sync_copy(zd_hbm, den_sp)

    def _mx(i, m):
        return jnp.maximum(m, n_v[pl.ds(i * L, L)])
    m = lax.fori_loop(0, N // L, _mx, jnp.full((L,), -jnp.inf, jnp.float32))
    lanes = lax.iota(jnp.int32, L)
    for sh in (8, 4, 2, 1):
        m = jnp.maximum(m, m[jnp.bitwise_xor(lanes, sh)])
    max_n = m

    plsc.subcore_barrier()

    zi = jnp.zeros((L,), jnp.int32)
    for b in range(RING):
        pltpu.async_copy(feat_hbm.at[rc_v[b] & CMASK],
                         bbuf.at[pl.ds(b * L, L)], gsems[b])
        pltpu.async_copy(sbuf.at[pl.ds(b * L, L)], out_sp.at[zi],
                         ssems[b], add=True)
        pltpu.async_copy(exst.at[b], den_sp.at[zi], dsems[b], add=True)

    def _group(o, b, prefetch):
        g = o * RING + b
        pk = rc_v[g]
        rg = pk >> CBITS
        cg = pk & CMASK
        pltpu.make_async_copy(
            feat_hbm.at[cg], bbuf.at[pl.ds(b * L, L)], gsems[b]).wait()
        sr = plsc.load_gather(s_v, [rg])
        nc_ = plsc.load_gather(n_v, [cg])
        e = sr + nc_
        e = jnp.where(e > 0, e, 0.2 * e)
        cap = sr + max_n
        cap = jnp.where(cap > 0, cap, 0.2 * cap)
        ex = jnp.exp(e - cap)

        pltpu.make_async_copy(exst.at[b], den_sp.at[rg], dsems[b]).wait()
        exst[b, :] = ex

        pltpu.make_async_copy(
            sbuf.at[pl.ds(b * L, L)], out_sp.at[rg], ssems[b]).wait()
        for j in range(L):
            w = ex[jnp.full((L,), j, jnp.int32)]
            for c in range(F // (2 * L)):
                pr = bbuf[b * L + j, pl.ds(c * L, L)]
                lo = plsc.bitcast(pr << 16, jnp.float32)
                hi = plsc.bitcast(pr & jnp.int32(-65536), jnp.float32)
                sbuf[b * L + j, pl.ds(2 * c * L, L)] = lo * w
                sbuf[b * L + j, pl.ds((2 * c + 1) * L, L)] = hi * w

        pltpu.async_copy(exst.at[b], den_sp.at[rg], dsems[b], add=True)
        pltpu.async_copy(sbuf.at[pl.ds(b * L, L)], out_sp.at[rg],
                         ssems[b], add=True)

        if prefetch:
            pg = g + RING
            pltpu.async_copy(feat_hbm.at[rc_v[pg] & CMASK],
                             bbuf.at[pl.ds(b * L, L)], gsems[b])

    def _outer(o, carry):
        for b in range(RING):
            _group(o, b, prefetch=True)
        return carry

    lax.fori_loop(0, NG // RING - 1, _outer, jnp.int32(0))
    for b in range(RING):
        _group(NG // RING - 1, b, prefetch=False)

    for b in range(RING):
        pltpu.make_async_copy(
            sbuf.at[pl.ds(b * L, L)], out_sp.at[zi], ssems[b]).wait()
        pltpu.make_async_copy(exst.at[b], den_sp.at[zi], dsems[b]).wait()

    plsc.subcore_barrier()

    pltpu.sync_copy(out_sp.at[pl.ds(tid * NPT, NPT)],
                    part_hbm.at[cid, pl.ds(tid * NPT, NPT)])

    @pl.when(tid == 0)
    def _():
        pltpu.sync_copy(den_sp, den_hbm.at[pl.ds(cid * N, N)])


def _sc_edges(feat, s, n, rc3d, zden):
    mesh = plsc.VectorSubcoreMesh(core_axis_name="c", subcore_axis_name="s")
    scratch = [
        pltpu.VMEM((N,), jnp.float32),
        pltpu.VMEM((N,), jnp.float32),
        pltpu.VMEM((NG, L), jnp.int32),
        pltpu.VMEM((RING * L, F // 2), jnp.int32),
        pltpu.VMEM((RING * L, F), jnp.float32),
        pltpu.VMEM((RING, L), jnp.float32),
        pltpu.VMEM_SHARED((N, F), jnp.float32),
        pltpu.VMEM_SHARED((N,), jnp.float32),
    ] + [pltpu.SemaphoreType.DMA] * (3 * RING)
    run = pl.kernel(
        _sc_body,
        out_type=[
            jax.ShapeDtypeStruct((NC, N, F), jnp.float32),
            jax.ShapeDtypeStruct((NC * N,), jnp.float32),
        ],
        mesh=mesh,
        scratch_types=scratch,
        compiler_params=pltpu.CompilerParams(
            needs_layout_passes=False, use_tc_tiling_on_sc=False),
    )
    return run(feat, s, n, rc3d, zden)


def _combine_body(p_ref, d_ref, b_ref, o_ref):
    ps = p_ref[0] + p_ref[1]
    den = d_ref[0] + d_ref[1]
    o_ref[...] = jnp.maximum(ps / (den[:, None] + 1e-9) + b_ref[...], 0.0)


def _combine(partials, denoms, bias2d):
    return pl.pallas_call(
        _combine_body,
        out_shape=jax.ShapeDtypeStruct((N, F), jnp.float32),
    )(partials, denoms, bias2d)


def kernel(X, edge_index, W, a_self, a_neigh, bias):
    e3d = edge_index.reshape(2, E // F, F)
    fb, s2, n2, rc = _dense(X, W, a_self, a_neigh, e3d)
    feat = jax.lax.bitcast_convert_type(
        fb[:, jnp.array(_PERM)].reshape(N, F // 2, 2), jnp.int32)
    s = s2.reshape(N)
    n = n2.reshape(N)
    rc3d = rc.reshape(NW, NG, L)
    zden = jnp.zeros((N,), jnp.float32)
    partials, denoms = _sc_edges(feat, s, n, rc3d, zden)
    return _combine(partials, denoms.reshape(NC, N), bias.reshape(1, F))
